# Initial kernel scaffold; baseline (speedup 1.0000x reference)
import jax
import jax.numpy as jnp
from jax.experimental import pallas as pl


def kernel(adj_indices, adj_values, inputs, W):
    raise NotImplementedError("write your pallas kernel here")


# speedup vs baseline: 5.6286x; 5.6286x over previous
import functools

import jax
import jax.numpy as jnp
from jax import lax
from jax.experimental import pallas as pl
from jax.experimental.pallas import tpu as pltpu
from jax.experimental.pallas import tpu_sc as plsc

B, N, F, E, D = 4, 10000, 128, 320000, 128

NUM_SC = 2
NUM_TILES = 16
EDGES_PER_TILE = E // NUM_TILES
KBLK = 80
NBLK = EDGES_PER_TILE // KBLK
ROWS_MAIN = 632
ROWS_LAST = N - 15 * ROWS_MAIN


def _mm_body(a_ref, w_ref, o_ref):
    o_ref[...] = jnp.dot(a_ref[...], w_ref[...],
                         preferred_element_type=jnp.float32)


def _tc_matmul(a, w):
    bm = 2000
    return pl.pallas_call(
        _mm_body,
        grid=(a.shape[0] // bm,),
        in_specs=[
            pl.BlockSpec((bm, F), lambda i: (i, 0)),
            pl.BlockSpec((F, D), lambda i: (0, 0)),
        ],
        out_specs=pl.BlockSpec((bm, D), lambda i: (i, 0)),
        out_shape=jax.ShapeDtypeStruct((a.shape[0], D), jnp.float32),
    )(a, w)


def _tanh_body(y_ref, o_ref):
    o_ref[...] = jnp.tanh(y_ref[...])


def _tc_tanh(y):
    bm = 2000
    return pl.pallas_call(
        _tanh_body,
        grid=(y.shape[0] // bm,),
        in_specs=[pl.BlockSpec((bm, D), lambda i: (i, 0))],
        out_specs=pl.BlockSpec((bm, D), lambda i: (i, 0)),
        out_shape=jax.ShapeDtypeStruct(y.shape, jnp.float32),
    )(y)


def _sc_body(x_hbm, rows_hbm, cols_hbm, vals_hbm, zeros_hbm, y_hbm,
             acc, g, idx_c, idx_r, vals_v):
    c = lax.axis_index("c")
    s = lax.axis_index("s")
    ebase = s * EDGES_PER_TILE
    rbase = s * ROWS_MAIN

    def rows_slice(fn):
        @pl.when(s < 15)
        def _():
            fn(ROWS_MAIN)

        @pl.when(s == 15)
        def _():
            fn(ROWS_LAST)

    def do_batch(b):
        rows_slice(lambda nr: pltpu.sync_copy(
            zeros_hbm.at[pl.ds(rbase, nr), :], acc.at[pl.ds(rbase, nr), :]))
        plsc.subcore_barrier()

        @pl.loop(0, NBLK)
        def _(i):
            off = ebase + i * KBLK
            pltpu.sync_copy(cols_hbm.at[pl.ds(off, KBLK)], idx_c)
            pltpu.sync_copy(rows_hbm.at[pl.ds(off, KBLK)], idx_r)
            pltpu.sync_copy(vals_hbm.at[pl.ds(off, KBLK)], vals_v)
            pltpu.sync_copy(x_hbm.at[b].at[idx_c], g)

            @pl.loop(0, KBLK, step=16)
            def _(k0):
                vv = vals_v[pl.ds(k0, 16)]
                for j in range(16):
                    v = vv[j]
                    for f in range(D // 16):
                        sl = pl.ds(f * 16, 16)
                        g[k0 + j, sl] = g[k0 + j, sl] * v

            pltpu.sync_copy(g, acc.at[idx_r], add=True)

        plsc.subcore_barrier()
        rows_slice(lambda nr: pltpu.sync_copy(
            acc.at[pl.ds(rbase, nr), :],
            y_hbm.at[b].at[pl.ds(rbase, nr), :]))

    @pl.when(c == 0)
    def _():
        do_batch(0)
        do_batch(1)

    @pl.when(c == 1)
    def _():
        do_batch(2)
        do_batch(3)


@jax.jit
def _sc_aggregate(x, rows, cols, vals):
    mesh = plsc.VectorSubcoreMesh(core_axis_name="c", subcore_axis_name="s")
    kern = pl.kernel(
        _sc_body,
        out_type=jax.ShapeDtypeStruct((B, N, D), jnp.float32),
        mesh=mesh,
        scratch_types=[
            pltpu.VMEM_SHARED((N, D), jnp.float32),
            pltpu.VMEM((KBLK, D), jnp.float32),
            pltpu.VMEM((KBLK,), jnp.int32),
            pltpu.VMEM((KBLK,), jnp.int32),
            pltpu.VMEM((KBLK,), jnp.float32),
        ],
    )
    zeros = jnp.zeros((N, D), jnp.float32)
    return kern(x, rows, cols, vals, zeros)


def kernel(adj_indices, adj_values, inputs, W):
    b, n, f = inputs.shape
    d = W.shape[1]
    rows = adj_indices[0].astype(jnp.int32)
    cols = adj_indices[1].astype(jnp.int32)
    x = _tc_matmul(inputs.reshape(b * n, f), W).reshape(b, n, d)
    y = _sc_aggregate(x, rows, cols, adj_values)
    out = _tc_tanh(y.reshape(b * n, d))
    return out.reshape(b, n, d)


# speedup vs baseline: 16.7742x; 2.9802x over previous
import jax
import jax.numpy as jnp
from jax import lax
from jax.experimental import pallas as pl
from jax.experimental.pallas import tpu as pltpu
from jax.experimental.pallas import tpu_sc as plsc

B, N, F, E, D = 4, 10000, 128, 320000, 128

NUM_TILES = 16
EDGES_PER_TILE = E // NUM_TILES
KBLK = 80
NBLK = EDGES_PER_TILE // KBLK
ROWS_MAIN = 632
ROWS_LAST = N - 15 * ROWS_MAIN
IDEPTH = 6


def _mm_body(a_ref, w_ref, o_ref):
    o_ref[...] = jnp.dot(a_ref[...], w_ref[...],
                         preferred_element_type=jnp.float32)


def _tc_matmul(a, w):
    bm = 2000
    return pl.pallas_call(
        _mm_body,
        grid=(a.shape[0] // bm,),
        in_specs=[
            pl.BlockSpec((bm, F), lambda i: (i, 0)),
            pl.BlockSpec((F, D), lambda i: (0, 0)),
        ],
        out_specs=pl.BlockSpec((bm, D), lambda i: (i, 0)),
        out_shape=jax.ShapeDtypeStruct((a.shape[0], D), jnp.float32),
    )(a, w)


def _tanh_body(y_ref, o_ref):
    o_ref[...] = jnp.tanh(y_ref[...])


def _tc_tanh(y):
    bm = 2000
    return pl.pallas_call(
        _tanh_body,
        grid=(y.shape[0] // bm,),
        in_specs=[pl.BlockSpec((bm, D), lambda i: (i, 0))],
        out_specs=pl.BlockSpec((bm, D), lambda i: (i, 0)),
        out_shape=jax.ShapeDtypeStruct(y.shape, jnp.float32),
    )(y)


def _sc_body(x_hbm, rows_hbm, cols_hbm, vals_hbm, zeros_hbm, y_hbm,
             acc, g0, g1, h0, h1, cv, rv, vv_r, gsem0, gsem1, ssem0, ssem1,
             isem):
    c = lax.axis_index("c")
    s = lax.axis_index("s")
    ebase = s * EDGES_PER_TILE
    rbase = s * ROWS_MAIN
    gbufs, gsems = (g0, g1), (gsem0, gsem1)
    hbufs, ssems = (h0, h1), (ssem0, ssem1)

    def rows_slice(fn):
        @pl.when(s < 15)
        def _():
            fn(ROWS_MAIN)

        @pl.when(s == 15)
        def _():
            fn(ROWS_LAST)

    def fetch(j, q):
        off = ebase + j * KBLK
        return (
            pltpu.make_async_copy(cols_hbm.at[pl.ds(off, KBLK)],
                                  cv.at[q], isem.at[q]),
            pltpu.make_async_copy(rows_hbm.at[pl.ds(off, KBLK)],
                                  rv.at[q], isem.at[q]),
            pltpu.make_async_copy(vals_hbm.at[pl.ds(off, KBLK)],
                                  vv_r.at[q], isem.at[q]),
        )

    @pl.loop(0, 2)
    def _(bi):
        b = 2 * c + bi
        xb = x_hbm.at[b]

        def gather(buf, q, sem):
            return pltpu.make_async_copy(xb.at[cv.at[q]], buf, sem)

        def scatter(buf, q, sem):
            return pltpu.make_async_copy(buf, acc.at[rv.at[q]], sem)

        rows_slice(lambda nr: pltpu.sync_copy(
            zeros_hbm.at[pl.ds(rbase, nr), :], acc.at[pl.ds(rbase, nr), :]))
        plsc.subcore_barrier()

        for j in range(4):
            for d_ in fetch(j, j):
                d_.start()
        for j in range(2):
            for d_ in fetch(j, j):
                d_.wait()
            gather(gbufs[j], j, gsems[j]).start()

        @pl.loop(0, NBLK, step=2)
        def _(i):
            for p in range(2):
                jb = i + p
                q = lax.rem(jb, IDEPTH)
                gather(gbufs[p], q, gsems[p]).wait()

                @pl.when(jb >= 2)
                def _():
                    scatter(hbufs[p], q, ssems[p]).wait()

                @pl.loop(0, KBLK, step=16)
                def _(k0):
                    vvec = vv_r[q, pl.ds(k0, 16)]
                    for j in range(16):
                        v = vvec[j]
                        for f in range(D // 16):
                            sl = pl.ds(f * 16, 16)
                            hbufs[p][k0 + j, sl] = gbufs[p][k0 + j, sl] * v

                scatter(hbufs[p], q, ssems[p]).start(add=True)

                @pl.when(jb + 2 < NBLK)
                def _():
                    q2 = lax.rem(jb + 2, IDEPTH)
                    for d_ in fetch(jb + 2, q2):
                        d_.wait()
                    gather(gbufs[p], q2, gsems[p]).start()

                @pl.when(jb + 4 < NBLK)
                def _():
                    q4 = lax.rem(jb + 4, IDEPTH)
                    for d_ in fetch(jb + 4, q4):
                        d_.start()

        scatter(h0, lax.rem(NBLK - 2, IDEPTH), ssem0).wait()
        scatter(h1, lax.rem(NBLK - 1, IDEPTH), ssem1).wait()

        plsc.subcore_barrier()
        rows_slice(lambda nr: pltpu.sync_copy(
            acc.at[pl.ds(rbase, nr), :],
            y_hbm.at[b].at[pl.ds(rbase, nr), :]))


@jax.jit
def _sc_aggregate(x, rows, cols, vals):
    mesh = plsc.VectorSubcoreMesh(core_axis_name="c", subcore_axis_name="s")
    kern = pl.kernel(
        _sc_body,
        out_type=jax.ShapeDtypeStruct((B, N, D), jnp.float32),
        mesh=mesh,
        scratch_types=[
            pltpu.VMEM_SHARED((N, D), jnp.float32),
            pltpu.VMEM((KBLK, D), jnp.float32),
            pltpu.VMEM((KBLK, D), jnp.float32),
            pltpu.VMEM((KBLK, D), jnp.float32),
            pltpu.VMEM((KBLK, D), jnp.float32),
            pltpu.VMEM((IDEPTH, KBLK), jnp.int32),
            pltpu.VMEM((IDEPTH, KBLK), jnp.int32),
            pltpu.VMEM((IDEPTH, KBLK), jnp.float32),
            pltpu.SemaphoreType.DMA,
            pltpu.SemaphoreType.DMA,
            pltpu.SemaphoreType.DMA,
            pltpu.SemaphoreType.DMA,
            pltpu.SemaphoreType.DMA((IDEPTH,)),
        ],
    )
    zeros = jnp.zeros((N, D), jnp.float32)
    return kern(x, rows, cols, vals, zeros)


def kernel(adj_indices, adj_values, inputs, W):
    b, n, f = inputs.shape
    d = W.shape[1]
    rows = adj_indices[0].astype(jnp.int32)
    cols = adj_indices[1].astype(jnp.int32)
    x = _tc_matmul(inputs.reshape(b * n, f), W).reshape(b, n, d)
    y = _sc_aggregate(x, rows, cols, adj_values)
    out = _tc_tanh(y.reshape(b * n, d))
    return out.reshape(b, n, d)
